# Initial kernel scaffold; baseline (speedup 1.0000x reference)
import jax
import jax.numpy as jnp
from jax.experimental import pallas as pl


def kernel(x, time_adj, W, b):
    raise NotImplementedError("write your pallas kernel here")


# speedup vs baseline: 1.0407x; 1.0407x over previous
import functools

import jax
import jax.numpy as jnp
from jax.experimental import pallas as pl
from jax.experimental.pallas import tpu as pltpu

_BM = 256


def _body(x_ref, w_ref, b_ref, a_ref, o_ref, h_ref):
    @pl.when(pl.program_id(0) == 0)
    def _():
        h = jax.lax.dot_general(
            x_ref[...], w_ref[...],
            dimension_numbers=(((1,), (1,)), ((), ())),
            preferred_element_type=jnp.float32,
        )
        h_ref[...] = (h + b_ref[...]).astype(jnp.bfloat16)

    a = a_ref[...].astype(jnp.bfloat16)
    o_ref[...] = jnp.dot(a, h_ref[...], preferred_element_type=jnp.float32)


@jax.jit
def kernel(x, time_adj, W, b):
    n, d_in = x.shape
    d_out = W.shape[0]
    b2 = b.reshape(1, d_out)
    grid = (pl.cdiv(n, _BM),)
    return pl.pallas_call(
        _body,
        grid=grid,
        in_specs=[
            pl.BlockSpec((n, d_in), lambda i: (0, 0)),
            pl.BlockSpec((d_out, d_in), lambda i: (0, 0)),
            pl.BlockSpec((1, d_out), lambda i: (0, 0)),
            pl.BlockSpec((_BM, n), lambda i: (i, 0)),
        ],
        out_specs=pl.BlockSpec((_BM, d_out), lambda i: (i, 0)),
        out_shape=jax.ShapeDtypeStruct((n, d_out), jnp.float32),
        scratch_shapes=[pltpu.VMEM((n, d_out), jnp.bfloat16)],
        compiler_params=pltpu.CompilerParams(
            dimension_semantics=("arbitrary",),
        ),
    )(x, W, b2, time_adj)
